# baseline (speedup 1.0000x reference)
import functools

import jax
import jax.numpy as jnp
from jax import lax
from jax.experimental import pallas as pl
from jax.experimental.pallas import tpu as pltpu
from jax.experimental.pallas import tpu_sc as plsc

H = 768
E = 64
EI = 96
N = 4096
B = 128
NBLK = N // B
T_MAX = NBLK + E - 1
P = T_MAX * B

RB = 512
NW = 32
RPW = N // NW


def _route_meta_body(x_ref, wr_ref, dst_ref, ie_ref, ib_ref, ni_ref,
                     ids_s, counts_s):
    t = pl.program_id(0)
    iota_e = lax.broadcasted_iota(jnp.int32, (E, 1), 0)

    @pl.when(t < NBLK)
    def _route():
        logits = lax.dot_general(wr_ref[...], x_ref[...],
                                 (((1,), (1,)), ((), ())),
                                 preferred_element_type=jnp.float32)
        m = jnp.max(logits, axis=0, keepdims=True)
        ids = jnp.min(jnp.where(logits == m, iota_e, E), axis=0,
                      keepdims=True).astype(jnp.int32)
        ids_s[pl.ds(t, 1), :] = ids
        oh = (ids == iota_e).astype(jnp.int32)

        @pl.when(t == 0)
        def _():
            counts_s[...] = jnp.zeros((E, 1), jnp.int32)

        counts_s[...] += jnp.sum(oh, axis=1, keepdims=True)

    @pl.when(t == NBLK)
    def _finalize():
        tlane = lax.broadcasted_iota(jnp.int32, (1, B), 1)
        tril = (lax.broadcasted_iota(jnp.int32, (B, 1), 0)
                < lax.broadcasted_iota(jnp.int32, (1, B), 1)
                ).astype(jnp.float32)
        tri_e = (lax.broadcasted_iota(jnp.int32, (E, 1), 0)
                 >= lax.broadcasted_iota(jnp.int32, (1, E), 1)
                 ).astype(jnp.float32)

        counts = counts_s[...]
        blocks_e = (counts + B - 1) // B
        cumblk = lax.dot_general(tri_e, blocks_e.astype(jnp.float32),
                                 (((1,), (0,)), ((), ())),
                                 preferred_element_type=jnp.float32
                                 ).astype(jnp.int32)
        cumblk_excl = cumblk - blocks_e
        pad_start = cumblk_excl * B
        ni = jnp.sum(blocks_e)

        ie_raw = jnp.sum((cumblk <= tlane).astype(jnp.int32), axis=0)
        tl = tlane[0]
        ie_at_last = jnp.sum(jnp.where(tl == ni - 1, ie_raw, 0))
        ie_ref[...] = jnp.where(tl < ni, ie_raw, ie_at_last)
        ib_ref[...] = jnp.minimum(tl, ni - 1)
        ni_ref[...] = jnp.full((B,), ni, jnp.int32)

        def body_b(i, counts2):
            oh = (ids_s[pl.ds(i, 1), :] == iota_e).astype(jnp.int32)
            ohf = oh.astype(jnp.float32)
            rank = lax.dot_general(ohf, tril, (((1,), (0,)), ((), ())),
                                   preferred_element_type=jnp.float32
                                   ).astype(jnp.int32)
            base = pad_start + counts2
            dst = jnp.sum(oh * (base + rank), axis=0)
            dst_ref[pl.ds(i, 1), :] = dst.reshape(1, B)
            return counts2 + jnp.sum(oh, axis=1, keepdims=True)

        lax.fori_loop(0, NBLK, body_b, jnp.zeros((E, 1), jnp.int32))


def _dispatch_meta(x2d, Wr):
    dst, ie, ib, ni = pl.pallas_call(
        _route_meta_body,
        grid=(NBLK + 1,),
        in_specs=[
            pl.BlockSpec((B, H), lambda t: (jnp.minimum(t, NBLK - 1), 0)),
            pl.BlockSpec((E, H), lambda t: (0, 0)),
        ],
        out_specs=[
            pl.BlockSpec((NBLK, B), lambda t: (0, 0)),
            pl.BlockSpec((B,), lambda t: (0,)),
            pl.BlockSpec((B,), lambda t: (0,)),
            pl.BlockSpec((B,), lambda t: (0,)),
        ],
        out_shape=[
            jax.ShapeDtypeStruct((NBLK, B), jnp.int32),
            jax.ShapeDtypeStruct((B,), jnp.int32),
            jax.ShapeDtypeStruct((B,), jnp.int32),
            jax.ShapeDtypeStruct((B,), jnp.int32),
        ],
        scratch_shapes=[
            pltpu.VMEM((NBLK, B), jnp.int32),
            pltpu.VMEM((E, 1), jnp.int32),
        ],
        compiler_params=pltpu.CompilerParams(
            dimension_semantics=("arbitrary",)),
    )(x2d, Wr)
    return ie, ib, ni, dst.reshape(N)


def _sc_wid():
    return lax.axis_index("s") * 2 + lax.axis_index("c")


@functools.cache
def _sc_permute_kernels():
    mesh = plsc.VectorSubcoreMesh(core_axis_name="c", subcore_axis_name="s")
    scratch = [
        pltpu.VMEM((RPW,), jnp.int32),
        pltpu.VMEM((RPW, H), jnp.float32),
        pltpu.SemaphoreType.DMA,
    ]

    @functools.partial(
        pl.kernel,
        out_type=jax.ShapeDtypeStruct((P, H), jnp.float32),
        mesh=mesh,
        scratch_types=scratch,
    )
    def sc_scatter(x_hbm, idx_hbm, out_hbm, idx_v, rows_v, sem):
        base = _sc_wid() * RPW
        pltpu.sync_copy(idx_hbm.at[pl.ds(base, RPW)], idx_v)
        pltpu.sync_copy(x_hbm.at[pl.ds(base, RPW)], rows_v)
        pltpu.async_copy(rows_v, out_hbm.at[idx_v], sem).wait()

    @functools.partial(
        pl.kernel,
        out_type=jax.ShapeDtypeStruct((N, H), jnp.float32),
        mesh=mesh,
        scratch_types=scratch,
    )
    def sc_gather(src_hbm, idx_hbm, out_hbm, idx_v, rows_v, sem):
        base = _sc_wid() * RPW
        pltpu.sync_copy(idx_hbm.at[pl.ds(base, RPW)], idx_v)
        pltpu.async_copy(src_hbm.at[idx_v], rows_v, sem).wait()
        pltpu.sync_copy(rows_v, out_hbm.at[pl.ds(base, RPW)])

    return sc_scatter, sc_gather


def _mlp_body(ie_ref, ib_ref, ni_ref, x_ref, wg_ref, wu_ref, wd_ref, o_ref):
    t = pl.program_id(0)

    @pl.when(t < ni_ref[0])
    def _():
        xb = x_ref[...]
        g = lax.dot_general(xb, wg_ref[...], (((1,), (1,)), ((), ())),
                            preferred_element_type=jnp.float32)
        u = lax.dot_general(xb, wu_ref[...], (((1,), (1,)), ((), ())),
                            preferred_element_type=jnp.float32)
        h = g * jax.nn.sigmoid(g) * u
        o_ref[...] = lax.dot_general(h, wd_ref[...], (((1,), (0,)), ((), ())),
                                     preferred_element_type=jnp.float32)


def _grouped_mlp(x_pad, Wg, Wu, Wd, item_expert, item_block, num_items):
    grid_spec = pltpu.PrefetchScalarGridSpec(
        num_scalar_prefetch=3,
        grid=(T_MAX,),
        in_specs=[
            pl.BlockSpec((B, H), lambda t, ie, ib, ni: (ib[t], 0)),
            pl.BlockSpec((EI, H), lambda t, ie, ib, ni: (ie[t], 0)),
            pl.BlockSpec((EI, H), lambda t, ie, ib, ni: (ie[t], 0)),
            pl.BlockSpec((EI, H), lambda t, ie, ib, ni: (ie[t], 0)),
        ],
        out_specs=pl.BlockSpec((B, H), lambda t, ie, ib, ni: (ib[t], 0)),
    )
    return pl.pallas_call(
        _mlp_body,
        grid_spec=grid_spec,
        out_shape=jax.ShapeDtypeStruct((P, H), jnp.float32),
        compiler_params=pltpu.CompilerParams(
            dimension_semantics=("arbitrary",)),
    )(item_expert, item_block, num_items, x_pad,
      Wg.reshape(E * EI, H), Wu.reshape(E * EI, H),
      jnp.swapaxes(Wd, 1, 2).reshape(E * EI, H))


def kernel(x, Wr, Wg, Wu, Wd):
    x2d = x.reshape(N, H)
    sc_scatter, sc_gather = _sc_permute_kernels()
    item_expert, item_block, num_items, dst_token = _dispatch_meta(x2d, Wr)
    x_pad = sc_scatter(x2d, dst_token)
    out_pad = _grouped_mlp(x_pad, Wg, Wu, Wd, item_expert, item_block,
                           num_items)
    out2d = sc_gather(out_pad, dst_token)
    return out2d.reshape(x.shape)


# speedup vs baseline: 1.0720x; 1.0720x over previous
import functools

import jax
import jax.numpy as jnp
from jax import lax
from jax.experimental import pallas as pl
from jax.experimental.pallas import tpu as pltpu
from jax.experimental.pallas import tpu_sc as plsc

H = 768
E = 64
EI = 96
N = 4096
B = 128
NBLK = N // B
T_MAX = NBLK + E - 1
P = T_MAX * B

RB = 512
NRB = N // RB
NW = 32
RPW = N // NW
NCH = 4
CH = RPW // NCH


def _route_meta_body(x_ref, wr_ref, dst_ref, ie_ref, ib_ref, ni_ref,
                     ids_s, counts_s):
    t = pl.program_id(0)
    iota_e = lax.broadcasted_iota(jnp.int32, (E, 1), 0)

    @pl.when(t < NRB)
    def _route():
        logits = lax.dot_general(wr_ref[...], x_ref[...],
                                 (((1,), (1,)), ((), ())),
                                 preferred_element_type=jnp.float32)
        m = jnp.max(logits, axis=0, keepdims=True)
        ids = jnp.min(jnp.where(logits == m,
                                lax.broadcasted_iota(jnp.int32, (E, RB), 0),
                                E), axis=0, keepdims=True)
        ids_s[pl.ds((RB // B) * t, RB // B), :] = ids.reshape(RB // B, B)
        oh = (ids == iota_e).astype(jnp.int32)

        @pl.when(t == 0)
        def _():
            counts_s[...] = jnp.zeros((E, 1), jnp.int32)

        counts_s[...] += jnp.sum(oh, axis=1, keepdims=True)

    @pl.when(t == NRB)
    def _finalize():
        tlane = lax.broadcasted_iota(jnp.int32, (1, B), 1)
        tril = (lax.broadcasted_iota(jnp.int32, (B, 1), 0)
                < lax.broadcasted_iota(jnp.int32, (1, B), 1)
                ).astype(jnp.float32)
        tri_e = (lax.broadcasted_iota(jnp.int32, (E, 1), 0)
                 >= lax.broadcasted_iota(jnp.int32, (1, E), 1)
                 ).astype(jnp.float32)

        counts = counts_s[...]
        blocks_e = (counts + B - 1) // B
        cumblk = lax.dot_general(tri_e, blocks_e.astype(jnp.float32),
                                 (((1,), (0,)), ((), ())),
                                 preferred_element_type=jnp.float32
                                 ).astype(jnp.int32)
        cumblk_excl = cumblk - blocks_e
        pad_start = cumblk_excl * B
        ni = jnp.sum(blocks_e)

        ie_raw = jnp.sum((cumblk <= tlane).astype(jnp.int32), axis=0)
        tl = tlane[0]
        ie_at_last = jnp.sum(jnp.where(tl == ni - 1, ie_raw, 0))
        ie_ref[...] = jnp.where(tl < ni, ie_raw, ie_at_last)
        ib_ref[...] = jnp.minimum(tl, ni - 1)
        ni_ref[...] = jnp.full((B,), ni, jnp.int32)

        def body_b(i, counts2):
            oh = (ids_s[pl.ds(i, 1), :] == iota_e).astype(jnp.int32)
            ohf = oh.astype(jnp.float32)
            rank = lax.dot_general(ohf, tril, (((1,), (0,)), ((), ())),
                                   preferred_element_type=jnp.float32
                                   ).astype(jnp.int32)
            base = pad_start + counts2
            dst = jnp.sum(oh * (base + rank), axis=0)
            dst_ref[pl.ds(i, 1), :] = dst.reshape(1, B)
            return counts2 + jnp.sum(oh, axis=1, keepdims=True)

        lax.fori_loop(0, NBLK, body_b, jnp.zeros((E, 1), jnp.int32))


def _dispatch_meta(x2d, Wr):
    dst, ie, ib, ni = pl.pallas_call(
        _route_meta_body,
        grid=(NRB + 1,),
        in_specs=[
            pl.BlockSpec((RB, H), lambda t: (jnp.minimum(t, NRB - 1), 0)),
            pl.BlockSpec((E, H), lambda t: (0, 0)),
        ],
        out_specs=[
            pl.BlockSpec((NBLK, B), lambda t: (0, 0)),
            pl.BlockSpec((B,), lambda t: (0,)),
            pl.BlockSpec((B,), lambda t: (0,)),
            pl.BlockSpec((B,), lambda t: (0,)),
        ],
        out_shape=[
            jax.ShapeDtypeStruct((NBLK, B), jnp.int32),
            jax.ShapeDtypeStruct((B,), jnp.int32),
            jax.ShapeDtypeStruct((B,), jnp.int32),
            jax.ShapeDtypeStruct((B,), jnp.int32),
        ],
        scratch_shapes=[
            pltpu.VMEM((NBLK, B), jnp.int32),
            pltpu.VMEM((E, 1), jnp.int32),
        ],
        compiler_params=pltpu.CompilerParams(
            dimension_semantics=("arbitrary",)),
    )(x2d, Wr)
    return ie, ib, ni, dst.reshape(N)


def _sc_wid():
    return lax.axis_index("s") * 2 + lax.axis_index("c")


@functools.cache
def _sc_permute_kernels():
    mesh = plsc.VectorSubcoreMesh(core_axis_name="c", subcore_axis_name="s")
    scratch = [
        pltpu.VMEM((NCH, CH), jnp.int32),
        pltpu.VMEM((NCH, CH, H), jnp.float32),
        pltpu.SemaphoreType.DMA((NCH,)),
        pltpu.SemaphoreType.DMA,
    ]

    @functools.partial(
        pl.kernel,
        out_type=jax.ShapeDtypeStruct((P, H), jnp.float32),
        mesh=mesh,
        scratch_types=scratch,
    )
    def sc_scatter(x_hbm, idx_hbm, out_hbm, idx_v, rows_v, sem_i, sem_s):
        wid = _sc_wid()
        base = wid * RPW
        pltpu.sync_copy(idx_hbm.at[wid], idx_v)
        scats = []
        for j in range(NCH):
            pltpu.sync_copy(x_hbm.at[pl.ds(base + j * CH, CH)], rows_v.at[j])
            scats.append(pltpu.async_copy(rows_v.at[j],
                                          out_hbm.at[idx_v.at[j]], sem_s))
        for c in scats:
            c.wait()

    @functools.partial(
        pl.kernel,
        out_type=jax.ShapeDtypeStruct((N, H), jnp.float32),
        mesh=mesh,
        scratch_types=scratch,
    )
    def sc_gather(src_hbm, idx_hbm, out_hbm, idx_v, rows_v, sem_i, sem_s):
        wid = _sc_wid()
        base = wid * RPW
        pltpu.sync_copy(idx_hbm.at[wid], idx_v)
        gats = [pltpu.async_copy(src_hbm.at[idx_v.at[j]], rows_v.at[j],
                                 sem_i.at[j])
                for j in range(NCH)]
        for j in range(NCH):
            gats[j].wait()
            pltpu.sync_copy(rows_v.at[j],
                            out_hbm.at[pl.ds(base + j * CH, CH)])

    return sc_scatter, sc_gather


def _mlp_body(ie_ref, ib_ref, ni_ref, x_ref, wg_ref, wu_ref, wd_ref, o_ref):
    t = pl.program_id(0)

    @pl.when(t < ni_ref[0])
    def _():
        xb = x_ref[...]
        g = lax.dot_general(xb, wg_ref[...], (((1,), (1,)), ((), ())),
                            preferred_element_type=jnp.float32)
        u = lax.dot_general(xb, wu_ref[...], (((1,), (1,)), ((), ())),
                            preferred_element_type=jnp.float32)
        h = g * jax.nn.sigmoid(g) * u
        o_ref[...] = lax.dot_general(h, wd_ref[...], (((1,), (0,)), ((), ())),
                                     preferred_element_type=jnp.float32)


def _grouped_mlp(x_pad, Wg, Wu, Wd, item_expert, item_block, num_items):
    grid_spec = pltpu.PrefetchScalarGridSpec(
        num_scalar_prefetch=3,
        grid=(T_MAX,),
        in_specs=[
            pl.BlockSpec((B, H), lambda t, ie, ib, ni: (ib[t], 0)),
            pl.BlockSpec((EI, H), lambda t, ie, ib, ni: (ie[t], 0)),
            pl.BlockSpec((EI, H), lambda t, ie, ib, ni: (ie[t], 0)),
            pl.BlockSpec((EI, H), lambda t, ie, ib, ni: (ie[t], 0)),
        ],
        out_specs=pl.BlockSpec((B, H), lambda t, ie, ib, ni: (ib[t], 0)),
    )
    return pl.pallas_call(
        _mlp_body,
        grid_spec=grid_spec,
        out_shape=jax.ShapeDtypeStruct((P, H), jnp.float32),
        compiler_params=pltpu.CompilerParams(
            dimension_semantics=("arbitrary",)),
    )(item_expert, item_block, num_items, x_pad,
      Wg.reshape(E * EI, H), Wu.reshape(E * EI, H),
      jnp.swapaxes(Wd, 1, 2).reshape(E * EI, H))


def kernel(x, Wr, Wg, Wu, Wd):
    x2d = x.reshape(N, H)
    sc_scatter, sc_gather = _sc_permute_kernels()
    item_expert, item_block, num_items, dst_token = _dispatch_meta(x2d, Wr)
    idx3 = dst_token.reshape(NW, NCH, CH)
    x_pad = sc_scatter(x2d, idx3)
    out_pad = _grouped_mlp(x_pad, Wg, Wu, Wd, item_expert, item_block,
                           num_items)
    out2d = sc_gather(out_pad, idx3)
    return out2d.reshape(x.shape)


# speedup vs baseline: 1.1468x; 1.0697x over previous
import functools

import jax
import jax.numpy as jnp
from jax import lax
from jax.experimental import pallas as pl
from jax.experimental.pallas import tpu as pltpu
from jax.experimental.pallas import tpu_sc as plsc

H = 768
E = 64
EI = 96
N = 4096
B = 96
T_MAX = -(-N // B) + E - 1
P = T_MAX * B

MB = 128
NMB = N // MB
TL = 128

RB = 512
NRB = N // RB
NW = 32
RPW = N // NW


def _route_meta_body(x_ref, wr_ref, dst_ref, ie_ref, ib_ref, ni_ref,
                     ids_s, counts_s):
    t = pl.program_id(0)
    iota_e = lax.broadcasted_iota(jnp.int32, (E, 1), 0)

    @pl.when(t < NRB)
    def _route():
        logits = lax.dot_general(wr_ref[...], x_ref[...],
                                 (((1,), (1,)), ((), ())),
                                 preferred_element_type=jnp.float32)
        m = jnp.max(logits, axis=0, keepdims=True)
        ids = jnp.min(jnp.where(logits == m,
                                lax.broadcasted_iota(jnp.int32, (E, RB), 0),
                                E), axis=0, keepdims=True)
        ids_s[pl.ds((RB // MB) * t, RB // MB), :] = ids.reshape(RB // MB, MB)
        oh = (ids == iota_e).astype(jnp.int32)

        @pl.when(t == 0)
        def _():
            counts_s[...] = jnp.zeros((E, 1), jnp.int32)

        counts_s[...] += jnp.sum(oh, axis=1, keepdims=True)

    @pl.when(t == NRB)
    def _finalize():
        tlane = lax.broadcasted_iota(jnp.int32, (1, TL), 1)
        tril = (lax.broadcasted_iota(jnp.int32, (MB, 1), 0)
                < lax.broadcasted_iota(jnp.int32, (1, MB), 1)
                ).astype(jnp.float32)
        tri_e = (lax.broadcasted_iota(jnp.int32, (E, 1), 0)
                 >= lax.broadcasted_iota(jnp.int32, (1, E), 1)
                 ).astype(jnp.float32)

        counts = counts_s[...]
        blocks_e = (counts + B - 1) // B
        cumblk = lax.dot_general(tri_e, blocks_e.astype(jnp.float32),
                                 (((1,), (0,)), ((), ())),
                                 preferred_element_type=jnp.float32
                                 ).astype(jnp.int32)
        cumblk_excl = cumblk - blocks_e
        pad_start = cumblk_excl * B
        ni = jnp.sum(blocks_e)

        ie_raw = jnp.sum((cumblk <= tlane).astype(jnp.int32), axis=0)
        tl = tlane[0]
        ie_at_last = jnp.sum(jnp.where(tl == ni - 1, ie_raw, 0))
        ie_ref[...] = jnp.where(tl < ni, ie_raw, ie_at_last)
        ib_ref[...] = jnp.minimum(tl, ni - 1)
        ni_ref[...] = jnp.full((TL,), ni, jnp.int32)

        def body_b(i, counts2):
            oh = (ids_s[pl.ds(i, 1), :] == iota_e).astype(jnp.int32)
            ohf = oh.astype(jnp.float32)
            rank = lax.dot_general(ohf, tril, (((1,), (0,)), ((), ())),
                                   preferred_element_type=jnp.float32
                                   ).astype(jnp.int32)
            base = pad_start + counts2
            dst = jnp.sum(oh * (base + rank), axis=0)
            dst_ref[pl.ds(i * MB, MB)] = dst
            return counts2 + jnp.sum(oh, axis=1, keepdims=True)

        lax.fori_loop(0, NMB, body_b, jnp.zeros((E, 1), jnp.int32))


def _dispatch_meta(x2d, Wr):
    dst, ie, ib, ni = pl.pallas_call(
        _route_meta_body,
        grid=(NRB + 1,),
        in_specs=[
            pl.BlockSpec((RB, H), lambda t: (jnp.minimum(t, NRB - 1), 0)),
            pl.BlockSpec((E, H), lambda t: (0, 0)),
        ],
        out_specs=[
            pl.BlockSpec((N,), lambda t: (0,)),
            pl.BlockSpec((TL,), lambda t: (0,)),
            pl.BlockSpec((TL,), lambda t: (0,)),
            pl.BlockSpec((TL,), lambda t: (0,)),
        ],
        out_shape=[
            jax.ShapeDtypeStruct((N,), jnp.int32),
            jax.ShapeDtypeStruct((TL,), jnp.int32),
            jax.ShapeDtypeStruct((TL,), jnp.int32),
            jax.ShapeDtypeStruct((TL,), jnp.int32),
        ],
        scratch_shapes=[
            pltpu.VMEM((NMB, MB), jnp.int32),
            pltpu.VMEM((E, 1), jnp.int32),
        ],
        compiler_params=pltpu.CompilerParams(
            dimension_semantics=("arbitrary",)),
    )(x2d, Wr)
    return ie, ib, ni, dst


def _sc_wid():
    return lax.axis_index("s") * 2 + lax.axis_index("c")


@functools.cache
def _sc_permute_kernels():
    mesh = plsc.VectorSubcoreMesh(core_axis_name="c", subcore_axis_name="s")
    scratch = [
        pltpu.VMEM((RPW,), jnp.int32),
        pltpu.VMEM((RPW, H), jnp.float32),
        pltpu.SemaphoreType.DMA,
    ]

    @functools.partial(
        pl.kernel,
        out_type=jax.ShapeDtypeStruct((P, H), jnp.float32),
        mesh=mesh,
        scratch_types=scratch,
    )
    def sc_scatter(x_hbm, idx_hbm, out_hbm, idx_v, rows_v, sem):
        base = _sc_wid() * RPW
        pltpu.sync_copy(idx_hbm.at[pl.ds(base, RPW)], idx_v)
        pltpu.sync_copy(x_hbm.at[pl.ds(base, RPW)], rows_v)
        pltpu.async_copy(rows_v, out_hbm.at[idx_v], sem).wait()

    @functools.partial(
        pl.kernel,
        out_type=jax.ShapeDtypeStruct((N, H), jnp.float32),
        mesh=mesh,
        scratch_types=scratch,
    )
    def sc_gather(src_hbm, idx_hbm, out_hbm, idx_v, rows_v, sem):
        base = _sc_wid() * RPW
        pltpu.sync_copy(idx_hbm.at[pl.ds(base, RPW)], idx_v)
        pltpu.async_copy(src_hbm.at[idx_v], rows_v, sem).wait()
        pltpu.sync_copy(rows_v, out_hbm.at[pl.ds(base, RPW)])

    return sc_scatter, sc_gather


def _mlp_body(ie_ref, ib_ref, ni_ref, x_ref, wg_ref, wu_ref, wd_ref, o_ref):
    t = pl.program_id(0)

    @pl.when(t < ni_ref[0])
    def _():
        xb = x_ref[...]
        g = lax.dot_general(xb, wg_ref[...], (((1,), (1,)), ((), ())),
                            preferred_element_type=jnp.float32)
        u = lax.dot_general(xb, wu_ref[...], (((1,), (1,)), ((), ())),
                            preferred_element_type=jnp.float32)
        h = g * jax.nn.sigmoid(g) * u
        o_ref[...] = lax.dot_general(h, wd_ref[...], (((1,), (0,)), ((), ())),
                                     preferred_element_type=jnp.float32)


def _grouped_mlp(x_pad, Wg, Wu, Wd, item_expert, item_block, num_items):
    grid_spec = pltpu.PrefetchScalarGridSpec(
        num_scalar_prefetch=3,
        grid=(T_MAX,),
        in_specs=[
            pl.BlockSpec((B, H), lambda t, ie, ib, ni: (ib[t], 0)),
            pl.BlockSpec((EI, H), lambda t, ie, ib, ni: (ie[t], 0)),
            pl.BlockSpec((EI, H), lambda t, ie, ib, ni: (ie[t], 0)),
            pl.BlockSpec((EI, H), lambda t, ie, ib, ni: (ie[t], 0)),
        ],
        out_specs=pl.BlockSpec((B, H), lambda t, ie, ib, ni: (ib[t], 0)),
    )
    return pl.pallas_call(
        _mlp_body,
        grid_spec=grid_spec,
        out_shape=jax.ShapeDtypeStruct((P, H), jnp.float32),
        compiler_params=pltpu.CompilerParams(
            dimension_semantics=("arbitrary",)),
    )(item_expert, item_block, num_items, x_pad,
      Wg.reshape(E * EI, H), Wu.reshape(E * EI, H),
      jnp.swapaxes(Wd, 1, 2).reshape(E * EI, H))


def kernel(x, Wr, Wg, Wu, Wd):
    x2d = x.reshape(N, H)
    sc_scatter, sc_gather = _sc_permute_kernels()
    item_expert, item_block, num_items, dst_token = _dispatch_meta(x2d, Wr)
    x_pad = sc_scatter(x2d, dst_token)
    out_pad = _grouped_mlp(x_pad, Wg, Wu, Wd, item_expert, item_block,
                           num_items)
    out2d = sc_gather(out_pad, dst_token)
    return out2d.reshape(x.shape)
